# baseline (speedup 1.0000x reference)
import functools

import jax
import jax.numpy as jnp
from jax.experimental import pallas as pl
from jax.experimental.pallas import tpu as pltpu


def _slice_kernel_body(ga_ref, gb_ref, gc_ref, axt_ref, guide_ref, out_ref,
                       *, gd, C, r):
    h = r // 2
    W = axt_ref.shape[1]
    ga = jnp.dot(ga_ref[0, 0], axt_ref[...],
                 preferred_element_type=jnp.float32).astype(jnp.bfloat16)
    gb = jnp.dot(gb_ref[0, 0], axt_ref[...],
                 preferred_element_type=jnp.float32).astype(jnp.bfloat16)
    gc = jnp.dot(gc_ref[0, 0], axt_ref[...],
                 preferred_element_type=jnp.float32).astype(jnp.bfloat16)
    du = gb - ga
    dl = gc - gb
    jv = jax.lax.broadcasted_iota(jnp.int32, (h, 1), 0).astype(jnp.float32)
    ju = (jv + jnp.float32(0.5)) * jnp.float32(1.0 / (2.0 * h))
    wyu = (ju + jnp.float32(0.5)).astype(jnp.bfloat16)
    wyl = ju.astype(jnp.bfloat16)
    g = guide_ref[0, 0]
    gz = jnp.clip(g * jnp.float32(gd) - jnp.float32(0.5),
                  jnp.float32(0.0), jnp.float32(gd - 1))
    wz = [jnp.maximum(jnp.float32(0.0),
                      jnp.float32(1.0) - jnp.abs(gz - jnp.float32(z))
                      ).astype(jnp.bfloat16)
          for z in range(gd)]
    for half, (g0, d, wy) in enumerate(((ga, du, wyu), (gb, dl, wyl))):
        lo = half * h
        for c in range(C):
            acc = jnp.zeros((h, W), dtype=jnp.bfloat16)
            for z in range(gd):
                row = z * C + c
                gyzc = g0[row][None, :] + wy * d[row][None, :]
                acc = acc + wz[z][lo:lo + h, :] * gyzc
            out_ref[0, c, lo:lo + h, :] = acc.astype(jnp.float32)


@jax.jit
def kernel(bilateral_grid, guidemap):
    B, C, gd, gh, gw = bilateral_grid.shape
    H, W = guidemap.shape[2], guidemap.shape[3]
    gt = jnp.transpose(bilateral_grid, (0, 3, 2, 1, 4)).reshape(B, gh, gd * C, gw)
    gt = gt.astype(jnp.bfloat16)

    pos = (jnp.arange(W, dtype=jnp.float32) + 0.5) * gw / W - 0.5
    f = jnp.floor(pos)
    wx = pos - f
    i0 = jnp.clip(f.astype(jnp.int32), 0, gw - 1)
    i1 = jnp.clip(f.astype(jnp.int32) + 1, 0, gw - 1)
    eye = jnp.eye(gw, dtype=jnp.float32)
    axt = (eye[i0] * (1.0 - wx)[:, None] + eye[i1] * wx[:, None]).T
    axt = axt.astype(jnp.bfloat16)

    r = 32

    def ya_map(b, i):
        return (b, jnp.clip(i - 1, 0, gh - 1), 0, 0)

    def yb_map(b, i):
        return (b, i, 0, 0)

    def yc_map(b, i):
        return (b, jnp.clip(i + 1, 0, gh - 1), 0, 0)

    body = functools.partial(_slice_kernel_body, gd=gd, C=C, r=r)
    return pl.pallas_call(
        body,
        grid=(B, H // r),
        in_specs=[
            pl.BlockSpec((1, 1, gd * C, gw), ya_map),
            pl.BlockSpec((1, 1, gd * C, gw), yb_map),
            pl.BlockSpec((1, 1, gd * C, gw), yc_map),
            pl.BlockSpec((gw, W), lambda b, i: (0, 0)),
            pl.BlockSpec((1, 1, r, W), lambda b, i: (b, 0, i, 0)),
        ],
        out_specs=pl.BlockSpec((1, C, r, W), lambda b, i: (b, 0, i, 0)),
        out_shape=jax.ShapeDtypeStruct((B, C, H, W), jnp.float32),
    )(gt, gt, gt, axt, guidemap)


# speedup vs baseline: 1.1496x; 1.1496x over previous
import functools

import jax
import jax.numpy as jnp
from jax.experimental import pallas as pl


def _slice_kernel_body(ga_ref, gb_ref, gc_ref, gd_ref, axt_ref, guide_ref,
                       out_ref, *, gd, C, r):
    h = r // 4
    W = axt_ref.shape[1]
    ga = jnp.dot(ga_ref[0, 0], axt_ref[...],
                 preferred_element_type=jnp.float32).astype(jnp.bfloat16)
    gb = jnp.dot(gb_ref[0, 0], axt_ref[...],
                 preferred_element_type=jnp.float32).astype(jnp.bfloat16)
    gc = jnp.dot(gc_ref[0, 0], axt_ref[...],
                 preferred_element_type=jnp.float32).astype(jnp.bfloat16)
    gdd = jnp.dot(gd_ref[0, 0], axt_ref[...],
                  preferred_element_type=jnp.float32).astype(jnp.bfloat16)
    dab = gb - ga
    dbc = gc - gb
    dcd = gdd - gc
    jv = jax.lax.broadcasted_iota(jnp.int32, (h, 1), 0).astype(jnp.float32)
    ju = (jv + jnp.float32(0.5)) * jnp.float32(1.0 / (2.0 * h))
    wyu = (ju + jnp.float32(0.5)).astype(jnp.bfloat16)
    wyl = ju.astype(jnp.bfloat16)
    g = guide_ref[0, 0]
    gz = jnp.clip(g * jnp.float32(gd) - jnp.float32(0.5),
                  jnp.float32(0.0), jnp.float32(gd - 1))
    wz = [jnp.maximum(jnp.float32(0.0),
                      jnp.float32(1.0) - jnp.abs(gz - jnp.float32(z))
                      ).astype(jnp.bfloat16)
          for z in range(gd)]
    quarters = ((ga, dab, wyu), (gb, dbc, wyl), (gb, dbc, wyu), (gc, dcd, wyl))
    for q, (g0, d, wy) in enumerate(quarters):
        lo = q * h
        for c in range(C):
            acc = None
            for z in range(gd):
                row = z * C + c
                gyzc = g0[row][None, :] + wy * d[row][None, :]
                term = wz[z][lo:lo + h, :] * gyzc
                acc = term if acc is None else acc + term
            out_ref[0, c, lo:lo + h, :] = acc.astype(jnp.float32)


@jax.jit
def kernel(bilateral_grid, guidemap):
    B, C, gd, gh, gw = bilateral_grid.shape
    H, W = guidemap.shape[2], guidemap.shape[3]
    gt = jnp.transpose(bilateral_grid, (0, 3, 2, 1, 4)).reshape(B, gh, gd * C, gw)
    gt = gt.astype(jnp.bfloat16)

    pos = (jnp.arange(W, dtype=jnp.float32) + 0.5) * gw / W - 0.5
    f = jnp.floor(pos)
    wx = pos - f
    i0 = jnp.clip(f.astype(jnp.int32), 0, gw - 1)
    i1 = jnp.clip(f.astype(jnp.int32) + 1, 0, gw - 1)
    eye = jnp.eye(gw, dtype=jnp.float32)
    axt = (eye[i0] * (1.0 - wx)[:, None] + eye[i1] * wx[:, None]).T
    axt = axt.astype(jnp.bfloat16)

    r = 64

    def ya_map(b, i):
        return (b, jnp.clip(2 * i - 1, 0, gh - 1), 0, 0)

    def yb_map(b, i):
        return (b, jnp.clip(2 * i, 0, gh - 1), 0, 0)

    def yc_map(b, i):
        return (b, jnp.clip(2 * i + 1, 0, gh - 1), 0, 0)

    def yd_map(b, i):
        return (b, jnp.clip(2 * i + 2, 0, gh - 1), 0, 0)

    body = functools.partial(_slice_kernel_body, gd=gd, C=C, r=r)
    return pl.pallas_call(
        body,
        grid=(B, H // r),
        in_specs=[
            pl.BlockSpec((1, 1, gd * C, gw), ya_map),
            pl.BlockSpec((1, 1, gd * C, gw), yb_map),
            pl.BlockSpec((1, 1, gd * C, gw), yc_map),
            pl.BlockSpec((1, 1, gd * C, gw), yd_map),
            pl.BlockSpec((gw, W), lambda b, i: (0, 0)),
            pl.BlockSpec((1, 1, r, W), lambda b, i: (b, 0, i, 0)),
        ],
        out_specs=pl.BlockSpec((1, C, r, W), lambda b, i: (b, 0, i, 0)),
        out_shape=jax.ShapeDtypeStruct((B, C, H, W), jnp.float32),
    )(gt, gt, gt, gt, axt, guidemap)


# speedup vs baseline: 1.2840x; 1.1169x over previous
import functools

import jax
import jax.numpy as jnp
from jax.experimental import pallas as pl


def _slice_kernel_body(ga_ref, gb_ref, gc_ref, gd_ref, axt_ref, guide_ref,
                       out_ref, *, gd, C, r):
    h = r // 4
    W = axt_ref.shape[1]
    ga = jnp.dot(ga_ref[0, 0], axt_ref[...],
                 preferred_element_type=jnp.float32).astype(jnp.bfloat16)
    gb = jnp.dot(gb_ref[0, 0], axt_ref[...],
                 preferred_element_type=jnp.float32).astype(jnp.bfloat16)
    gc = jnp.dot(gc_ref[0, 0], axt_ref[...],
                 preferred_element_type=jnp.float32).astype(jnp.bfloat16)
    gdd = jnp.dot(gd_ref[0, 0], axt_ref[...],
                  preferred_element_type=jnp.float32).astype(jnp.bfloat16)
    jv = jax.lax.broadcasted_iota(jnp.int32, (h, 1), 0).astype(jnp.float32)
    ju = (jv + jnp.float32(0.5)) * jnp.float32(1.0 / (2.0 * h))
    wyu = (ju + jnp.float32(0.5)).astype(jnp.bfloat16)
    wyl = ju.astype(jnp.bfloat16)
    g = guide_ref[0, 0]
    gz = jnp.clip(g * jnp.float32(gd) - jnp.float32(0.5),
                  jnp.float32(0.0), jnp.float32(gd - 1))
    one = jnp.bfloat16(1.0)
    zero = jnp.bfloat16(0.0)
    wz = [jnp.maximum(zero, one - jnp.abs(
              (gz - jnp.float32(z)).astype(jnp.bfloat16)))
          for z in range(gd)]
    quarters = ((ga, gb, wyu), (gb, gc, wyl), (gb, gc, wyu), (gc, gdd, wyl))
    for q, (g0, g1, wy) in enumerate(quarters):
        lo = q * h
        vq = [wz[z][lo:lo + h, :] * wy for z in range(gd)]
        uq = [wz[z][lo:lo + h, :] - vq[z] for z in range(gd)]
        for c in range(C):
            acc = None
            for z in range(gd):
                row = z * C + c
                term = uq[z] * g0[row][None, :] + vq[z] * g1[row][None, :]
                acc = term if acc is None else acc + term
            out_ref[0, c, lo:lo + h, :] = acc.astype(jnp.float32)


@jax.jit
def kernel(bilateral_grid, guidemap):
    B, C, gd, gh, gw = bilateral_grid.shape
    H, W = guidemap.shape[2], guidemap.shape[3]
    gt = jnp.transpose(bilateral_grid, (0, 3, 2, 1, 4)).reshape(B, gh, gd * C, gw)
    gt = gt.astype(jnp.bfloat16)

    pos = (jnp.arange(W, dtype=jnp.float32) + 0.5) * gw / W - 0.5
    f = jnp.floor(pos)
    wx = pos - f
    i0 = jnp.clip(f.astype(jnp.int32), 0, gw - 1)
    i1 = jnp.clip(f.astype(jnp.int32) + 1, 0, gw - 1)
    eye = jnp.eye(gw, dtype=jnp.float32)
    axt = (eye[i0] * (1.0 - wx)[:, None] + eye[i1] * wx[:, None]).T
    axt = axt.astype(jnp.bfloat16)

    r = 64

    def ya_map(b, i):
        return (b, jnp.clip(2 * i - 1, 0, gh - 1), 0, 0)

    def yb_map(b, i):
        return (b, jnp.clip(2 * i, 0, gh - 1), 0, 0)

    def yc_map(b, i):
        return (b, jnp.clip(2 * i + 1, 0, gh - 1), 0, 0)

    def yd_map(b, i):
        return (b, jnp.clip(2 * i + 2, 0, gh - 1), 0, 0)

    body = functools.partial(_slice_kernel_body, gd=gd, C=C, r=r)
    return pl.pallas_call(
        body,
        grid=(B, H // r),
        in_specs=[
            pl.BlockSpec((1, 1, gd * C, gw), ya_map),
            pl.BlockSpec((1, 1, gd * C, gw), yb_map),
            pl.BlockSpec((1, 1, gd * C, gw), yc_map),
            pl.BlockSpec((1, 1, gd * C, gw), yd_map),
            pl.BlockSpec((gw, W), lambda b, i: (0, 0)),
            pl.BlockSpec((1, 1, r, W), lambda b, i: (b, 0, i, 0)),
        ],
        out_specs=pl.BlockSpec((1, C, r, W), lambda b, i: (b, 0, i, 0)),
        out_shape=jax.ShapeDtypeStruct((B, C, H, W), jnp.float32),
    )(gt, gt, gt, gt, axt, guidemap)


# speedup vs baseline: 1.3696x; 1.0666x over previous
import functools

import jax
import jax.numpy as jnp
from jax.experimental import pallas as pl


def _slice_kernel_body(*refs, gd, C, r):
    nrows = r // 32 + 2
    grow_refs = refs[:nrows]
    axt_ref, guide_ref, out_ref = refs[nrows], refs[nrows + 1], refs[nrows + 2]
    h = 16
    W = axt_ref.shape[1]
    grows = [jnp.dot(gr[0, 0], axt_ref[...],
                     preferred_element_type=jnp.float32).astype(jnp.bfloat16)
             for gr in grow_refs]
    jv = jax.lax.broadcasted_iota(jnp.int32, (h, 1), 0).astype(jnp.float32)
    ju = (jv + jnp.float32(0.5)) * jnp.float32(1.0 / (2.0 * h))
    wyu = (ju + jnp.float32(0.5)).astype(jnp.bfloat16)
    wyl = ju.astype(jnp.bfloat16)
    g = guide_ref[0, 0]
    gz = jnp.clip(g * jnp.float32(gd) - jnp.float32(0.5),
                  jnp.float32(0.0), jnp.float32(gd - 1))
    one = jnp.bfloat16(1.0)
    zero = jnp.bfloat16(0.0)
    wz = [jnp.maximum(zero, one - jnp.abs(
              (gz - jnp.float32(z)).astype(jnp.bfloat16)))
          for z in range(gd)]
    quarters = [(grows[(q + 1) // 2], grows[(q + 1) // 2 + 1],
                 wyu if q % 2 == 0 else wyl) for q in range(r // 16)]
    for q, (g0, g1, wy) in enumerate(quarters):
        lo = q * h
        vq = [wz[z][lo:lo + h, :] * wy for z in range(gd)]
        uq = [wz[z][lo:lo + h, :] - vq[z] for z in range(gd)]
        for c in range(C):
            acc = None
            for z in range(gd):
                row = z * C + c
                term = uq[z] * g0[row][None, :] + vq[z] * g1[row][None, :]
                acc = term if acc is None else acc + term
            out_ref[0, c, lo:lo + h, :] = acc.astype(jnp.float32)


@jax.jit
def kernel(bilateral_grid, guidemap):
    B, C, gd, gh, gw = bilateral_grid.shape
    H, W = guidemap.shape[2], guidemap.shape[3]
    gt = jnp.transpose(bilateral_grid, (0, 3, 2, 1, 4)).reshape(B, gh, gd * C, gw)
    gt = gt.astype(jnp.bfloat16)

    pos = (jnp.arange(W, dtype=jnp.float32) + 0.5) * gw / W - 0.5
    f = jnp.floor(pos)
    wx = pos - f
    i0 = jnp.clip(f.astype(jnp.int32), 0, gw - 1)
    i1 = jnp.clip(f.astype(jnp.int32) + 1, 0, gw - 1)
    eye = jnp.eye(gw, dtype=jnp.float32)
    axt = (eye[i0] * (1.0 - wx)[:, None] + eye[i1] * wx[:, None]).T
    axt = axt.astype(jnp.bfloat16)

    r = 128
    nrows = r // 32 + 2

    def yk_map(k):
        def m(b, i):
            return (b, jnp.clip((r // 32) * i + k - 1, 0, gh - 1), 0, 0)
        return m

    body = functools.partial(_slice_kernel_body, gd=gd, C=C, r=r)
    return pl.pallas_call(
        body,
        grid=(B, H // r),
        in_specs=(
            [pl.BlockSpec((1, 1, gd * C, gw), yk_map(k)) for k in range(nrows)]
            + [pl.BlockSpec((gw, W), lambda b, i: (0, 0)),
               pl.BlockSpec((1, 1, r, W), lambda b, i: (b, 0, i, 0))]
        ),
        out_specs=pl.BlockSpec((1, C, r, W), lambda b, i: (b, 0, i, 0)),
        out_shape=jax.ShapeDtypeStruct((B, C, H, W), jnp.float32),
    )(*([gt] * nrows), axt, guidemap)


# speedup vs baseline: 1.4004x; 1.0226x over previous
import functools

import jax
import jax.numpy as jnp
from jax.experimental import pallas as pl


def _slice_kernel_body(*refs, gd, C, r):
    nrows = r // 32 + 2
    grow_refs = refs[:nrows]
    axt_ref, guide_ref, out_ref = refs[nrows], refs[nrows + 1], refs[nrows + 2]
    h = 16
    W = axt_ref.shape[1]
    grows = [jnp.dot(gr[0, 0], axt_ref[...],
                     preferred_element_type=jnp.float32).astype(jnp.bfloat16)
             for gr in grow_refs]
    jv = jax.lax.broadcasted_iota(jnp.int32, (h, 1), 0).astype(jnp.float32)
    ju = (jv + jnp.float32(0.5)) * jnp.float32(1.0 / (2.0 * h))
    wyu = (ju + jnp.float32(0.5)).astype(jnp.bfloat16)
    wyl = ju.astype(jnp.bfloat16)
    g = guide_ref[0, 0]
    gz = jnp.clip(g * jnp.float32(gd) - jnp.float32(0.5),
                  jnp.float32(0.0), jnp.float32(gd - 1))
    one = jnp.bfloat16(1.0)
    zero = jnp.bfloat16(0.0)
    wz = [jnp.maximum(zero, one - jnp.abs(
              (gz - jnp.float32(z)).astype(jnp.bfloat16)))
          for z in range(gd)]
    quarters = [(grows[(q + 1) // 2], grows[(q + 1) // 2 + 1],
                 wyu if q % 2 == 0 else wyl) for q in range(r // 16)]
    for q, (g0, g1, wy) in enumerate(quarters):
        lo = q * h
        vq = [wz[z][lo:lo + h, :] * wy for z in range(gd)]
        uq = [wz[z][lo:lo + h, :] - vq[z] for z in range(gd)]
        for c in range(C):
            acc = None
            for z in range(gd):
                row = z * C + c
                term = uq[z] * g0[row][None, :] + vq[z] * g1[row][None, :]
                acc = term if acc is None else acc + term
            out_ref[0, c, lo:lo + h, :] = acc.astype(jnp.float32)


@jax.jit
def kernel(bilateral_grid, guidemap):
    B, C, gd, gh, gw = bilateral_grid.shape
    H, W = guidemap.shape[2], guidemap.shape[3]
    gt = jnp.transpose(bilateral_grid, (0, 3, 2, 1, 4)).reshape(B, gh, gd * C, gw)
    gt = gt.astype(jnp.bfloat16)

    pos = (jnp.arange(W, dtype=jnp.float32) + 0.5) * gw / W - 0.5
    f = jnp.floor(pos)
    wx = pos - f
    i0 = jnp.clip(f.astype(jnp.int32), 0, gw - 1)
    i1 = jnp.clip(f.astype(jnp.int32) + 1, 0, gw - 1)
    eye = jnp.eye(gw, dtype=jnp.float32)
    axt = (eye[i0] * (1.0 - wx)[:, None] + eye[i1] * wx[:, None]).T
    axt = axt.astype(jnp.bfloat16)

    r = 256
    nrows = r // 32 + 2

    def yk_map(k):
        def m(b, i):
            return (b, jnp.clip((r // 32) * i + k - 1, 0, gh - 1), 0, 0)
        return m

    body = functools.partial(_slice_kernel_body, gd=gd, C=C, r=r)
    return pl.pallas_call(
        body,
        grid=(B, H // r),
        in_specs=(
            [pl.BlockSpec((1, 1, gd * C, gw), yk_map(k)) for k in range(nrows)]
            + [pl.BlockSpec((gw, W), lambda b, i: (0, 0)),
               pl.BlockSpec((1, 1, r, W), lambda b, i: (b, 0, i, 0))]
        ),
        out_specs=pl.BlockSpec((1, C, r, W), lambda b, i: (b, 0, i, 0)),
        out_shape=jax.ShapeDtypeStruct((B, C, H, W), jnp.float32),
    )(*([gt] * nrows), axt, guidemap)
